# baseline (speedup 1.0000x reference)
import numpy as np
import jax
import jax.numpy as jnp
from jax.experimental import pallas as pl
from jax.experimental.pallas import tpu as pltpu

_B = 512
_O = 5
_OBS = 128
_HID = 256
_EPN = _O - 1
_F = 4 * _HID
_EPS = _O * _EPN

_BB = 128
_R = _BB * _O
_RE = _BB * _EPS
_NH = 4
_BH = _BB // _NH
_RH = _BH * _O


def _sel_matrices():
    s0 = np.zeros((_EPS, _O), np.float32)
    c0 = np.zeros((_EPS, _O), np.float32)
    e = 0
    for i in range(_O):
        for j in range(_O):
            if i == j:
                continue
            s0[e, i] = 1.0
            c0[e, j] = 1.0
            e += 1
    eye = np.eye(_BH, dtype=np.float32)
    S = np.concatenate([np.kron(eye, s0), np.kron(eye, c0)], axis=1)
    Ragg = np.kron(eye, s0.T)
    return S, Ragg


_S_NP, _RAGG_NP = _sel_matrices()


def _c4_flat(W):
    Wh = W.astype(jnp.bfloat16)
    blocks = [Wh[k] for k in range(4)]
    rows = [jnp.concatenate([blocks[(g - h) % 4] for g in range(4)], axis=1)
            for h in range(4)]
    return jnp.concatenate(rows, axis=0)


def _ln_relu(y, gamma, beta):
    outs = []
    for gi in range(4):
        c = y[:, gi * _HID:(gi + 1) * _HID]
        mu = jnp.mean(c, axis=1, keepdims=True)
        d = c - mu
        var = jnp.mean(d * d, axis=1, keepdims=True)
        outs.append(d * jax.lax.rsqrt(var + 1e-5) * gamma + beta)
    return jnp.maximum(jnp.concatenate(outs, axis=1), 0.0)


def _dot(a, b):
    return jnp.dot(a.astype(jnp.bfloat16), b,
                   preferred_element_type=jnp.float32)


def _chain(x, M, sel, ragg, w1s, w1t, b1, w2, b2, ge, bne, w3, b3,
           wn1o, wav, wn1a, bn1, wn2, bn2, gn, bnn, wn3, bn3):
    A = _dot(x, w1s)
    T = _dot(x, w1t)
    AT = jnp.concatenate([A, T], axis=0)

    e = _dot(sel, AT) + b1
    e = jnp.maximum(e, 0.0)

    e = _dot(e, w2) + b2
    e = _ln_relu(e, ge, bne)
    e = _dot(e, w3) + b3

    agg = _dot(ragg, e)

    n = _dot(x, wn1o) + _dot(M, wav) + _dot(agg, wn1a) + bn1
    n = jnp.maximum(n, 0.0)
    n = _dot(n, wn2) + bn2
    n = _ln_relu(n, gn, bnn)
    return _dot(n, wn3) + bn3


def _body(x_ref, act_ref, *refs):
    out_ref = refs[-1]
    ws = tuple(r[...] for r in refs[:-1])

    act = act_ref[0, 0, :].reshape(_R, 1)
    rr = jax.lax.broadcasted_iota(jnp.int32, (_R, 4), 0)
    hh = jax.lax.broadcasted_iota(jnp.int32, (_R, 4), 1)
    M = (act == 4 * (rr % _O) + hh).astype(jnp.float32)

    for h in range(_NH):
        r0 = h * _RH
        out_ref[r0:r0 + _RH, :] = _chain(
            x_ref[r0:r0 + _RH, :], M[r0:r0 + _RH, :], *ws)


def kernel(states, action, We1, be1, We2, be2, ge, bne, We3, be3,
           Wn1, bn1, Wn2, bn2, gn, bnn, Wn3, bn3):
    x = states.reshape(_B * _O, 4 * _OBS)
    nblk = _B // _BB
    act = jnp.repeat(action.astype(jnp.int32), _O).reshape(nblk, 1, _R)

    bf16 = jnp.bfloat16
    S = jnp.asarray(_S_NP, dtype=bf16)
    Ragg = jnp.asarray(_RAGG_NP, dtype=bf16)
    W1s = _c4_flat(We1[:, :_OBS, :]).astype(bf16)
    W1t = _c4_flat(We1[:, _OBS:, :]).astype(bf16)
    W2 = _c4_flat(We2).astype(bf16)
    W3 = _c4_flat(We3).astype(bf16)
    Wn1o = _c4_flat(Wn1[:, :_OBS, :]).astype(bf16)
    Wav = _c4_flat(Wn1[:, _OBS:_OBS + 1, :]).astype(bf16)
    Wn1a = _c4_flat(Wn1[:, _OBS + 1:, :]).astype(bf16)
    Wn2f = _c4_flat(Wn2).astype(bf16)
    Wn3f = _c4_flat(Wn3).astype(bf16)

    b1 = jnp.tile(be1, 4).reshape(1, _F)
    b2 = jnp.tile(be2, 4).reshape(1, _F)
    b3 = jnp.tile(be3, 4).reshape(1, _F)
    bn1r = jnp.tile(bn1, 4).reshape(1, _F)
    bn2r = jnp.tile(bn2, 4).reshape(1, _F)
    bn3r = jnp.tile(bn3, 4).reshape(1, 4 * _OBS)
    ge2 = ge.reshape(1, _HID)
    bne2 = bne.reshape(1, _HID)
    gn2 = gn.reshape(1, _HID)
    bnn2 = bnn.reshape(1, _HID)

    def const_spec(a):
        nd = a.ndim
        return pl.BlockSpec(a.shape, lambda i, _nd=nd: (0,) * _nd)

    weights = (S, Ragg, W1s, W1t, b1, W2, b2, ge2, bne2, W3, b3,
               Wn1o, Wav, Wn1a, bn1r, Wn2f, bn2r, gn2, bnn2, Wn3f, bn3r)

    out = pl.pallas_call(
        _body,
        grid=(nblk,),
        in_specs=[
            pl.BlockSpec((_R, 4 * _OBS), lambda i: (i, 0)),
            pl.BlockSpec((1, 1, _R), lambda i: (i, 0, 0)),
        ] + [const_spec(w) for w in weights],
        out_specs=pl.BlockSpec((_R, 4 * _OBS), lambda i: (i, 0)),
        out_shape=jax.ShapeDtypeStruct((_B * _O, 4 * _OBS), jnp.float32),
        compiler_params=pltpu.CompilerParams(
            dimension_semantics=("arbitrary",)),
    )(x, act, *weights)

    return out.reshape(_B, _O, 4, _OBS)


# speedup vs baseline: 2.1526x; 2.1526x over previous
import numpy as np
import jax
import jax.numpy as jnp
from jax.experimental import pallas as pl
from jax.experimental.pallas import tpu as pltpu

_B = 512
_O = 5
_OBS = 128
_HID = 256
_EPN = _O - 1
_F = 4 * _HID
_EPS = _O * _EPN

_BB = 128
_R = _BB * _O
_RE = _BB * _EPS
_NH = 4
_BH = _BB // _NH
_RH = _BH * _O


def _sel_matrices():
    s0 = np.zeros((_EPS, _O), np.float32)
    c0 = np.zeros((_EPS, _O), np.float32)
    e = 0
    for i in range(_O):
        for j in range(_O):
            if i == j:
                continue
            s0[e, i] = 1.0
            c0[e, j] = 1.0
            e += 1
    eye = np.eye(_BH, dtype=np.float32)
    S = np.concatenate([np.kron(eye, s0), np.kron(eye, c0)], axis=1)
    Ragg = np.kron(eye, s0.T)
    return S, Ragg


_S_NP, _RAGG_NP = _sel_matrices()


def _c4_flat(W):
    g = jnp.arange(4)[:, None]
    h = jnp.arange(4)[None, :]
    Wfull = W.astype(jnp.bfloat16)[(g - h) % 4]
    Wt = jnp.transpose(Wfull, (1, 2, 0, 3))
    return Wt.reshape(4 * W.shape[1], 4 * W.shape[2])


def _ln_relu(y, gamma, beta):
    outs = []
    for gi in range(4):
        c = y[:, gi * _HID:(gi + 1) * _HID]
        mu = jnp.mean(c, axis=1, keepdims=True)
        d = c - mu
        var = jnp.mean(d * d, axis=1, keepdims=True)
        outs.append(d * jax.lax.rsqrt(var + 1e-5) * gamma + beta)
    return jnp.maximum(jnp.concatenate(outs, axis=1), 0.0)


def _dot(a, b):
    return jnp.dot(a.astype(jnp.bfloat16), b,
                   preferred_element_type=jnp.float32)


def _chain(x, M, sel, ragg, w1s, w1t, b1, w2, b2, ge, bne, w3, b3,
           wn1o, wav, wn1a, bn1, wn2, bn2, gn, bnn, wn3, bn3):
    A = _dot(x, w1s)
    T = _dot(x, w1t)
    AT = jnp.concatenate([A, T], axis=0)

    e = _dot(sel, AT) + b1
    e = jnp.maximum(e, 0.0)

    e = _dot(e, w2) + b2
    e = _ln_relu(e, ge, bne)
    e = _dot(e, w3) + b3

    agg = _dot(ragg, e)

    n = _dot(x, wn1o) + _dot(M, wav) + _dot(agg, wn1a) + bn1
    n = jnp.maximum(n, 0.0)
    n = _dot(n, wn2) + bn2
    n = _ln_relu(n, gn, bnn)
    return _dot(n, wn3) + bn3


def _body(x_ref, act_ref, *refs):
    out_ref = refs[-1]
    ws = tuple(r[...] for r in refs[:-1])

    act = act_ref[0, 0, :].reshape(_R, 1)
    rr = jax.lax.broadcasted_iota(jnp.int32, (_R, 4), 0)
    hh = jax.lax.broadcasted_iota(jnp.int32, (_R, 4), 1)
    M = (act == 4 * (rr % _O) + hh).astype(jnp.float32)

    for h in range(_NH):
        r0 = h * _RH
        out_ref[r0:r0 + _RH, :] = _chain(
            x_ref[r0:r0 + _RH, :], M[r0:r0 + _RH, :], *ws)


def kernel(states, action, We1, be1, We2, be2, ge, bne, We3, be3,
           Wn1, bn1, Wn2, bn2, gn, bnn, Wn3, bn3):
    x = states.reshape(_B * _O, 4 * _OBS)
    nblk = _B // _BB
    act = jnp.repeat(action.astype(jnp.int32), _O).reshape(nblk, 1, _R)

    bf16 = jnp.bfloat16
    S = jnp.asarray(_S_NP, dtype=bf16)
    Ragg = jnp.asarray(_RAGG_NP, dtype=bf16)
    W1s = _c4_flat(We1[:, :_OBS, :]).astype(bf16)
    W1t = _c4_flat(We1[:, _OBS:, :]).astype(bf16)
    W2 = _c4_flat(We2).astype(bf16)
    W3 = _c4_flat(We3).astype(bf16)
    Wn1o = _c4_flat(Wn1[:, :_OBS, :]).astype(bf16)
    Wav = _c4_flat(Wn1[:, _OBS:_OBS + 1, :]).astype(bf16)
    Wn1a = _c4_flat(Wn1[:, _OBS + 1:, :]).astype(bf16)
    Wn2f = _c4_flat(Wn2).astype(bf16)
    Wn3f = _c4_flat(Wn3).astype(bf16)

    b1 = jnp.tile(be1, 4).reshape(1, _F)
    b2 = jnp.tile(be2, 4).reshape(1, _F)
    b3 = jnp.tile(be3, 4).reshape(1, _F)
    bn1r = jnp.tile(bn1, 4).reshape(1, _F)
    bn2r = jnp.tile(bn2, 4).reshape(1, _F)
    bn3r = jnp.tile(bn3, 4).reshape(1, 4 * _OBS)
    ge2 = ge.reshape(1, _HID)
    bne2 = bne.reshape(1, _HID)
    gn2 = gn.reshape(1, _HID)
    bnn2 = bnn.reshape(1, _HID)

    def const_spec(a):
        nd = a.ndim
        return pl.BlockSpec(a.shape, lambda i, _nd=nd: (0,) * _nd)

    weights = (S, Ragg, W1s, W1t, b1, W2, b2, ge2, bne2, W3, b3,
               Wn1o, Wav, Wn1a, bn1r, Wn2f, bn2r, gn2, bnn2, Wn3f, bn3r)

    out = pl.pallas_call(
        _body,
        grid=(nblk,),
        in_specs=[
            pl.BlockSpec((_R, 4 * _OBS), lambda i: (i, 0)),
            pl.BlockSpec((1, 1, _R), lambda i: (i, 0, 0)),
        ] + [const_spec(w) for w in weights],
        out_specs=pl.BlockSpec((_R, 4 * _OBS), lambda i: (i, 0)),
        out_shape=jax.ShapeDtypeStruct((_B * _O, 4 * _OBS), jnp.float32),
        compiler_params=pltpu.CompilerParams(
            dimension_semantics=("arbitrary",)),
    )(x, act, *weights)

    return out.reshape(_B, _O, 4, _OBS)


# speedup vs baseline: 2.1830x; 1.0141x over previous
import numpy as np
import jax
import jax.numpy as jnp
from jax.experimental import pallas as pl
from jax.experimental.pallas import tpu as pltpu

_B = 512
_O = 5
_OBS = 128
_HID = 256
_EPN = _O - 1
_F = 4 * _HID
_EPS = _O * _EPN

_BB = 128
_R = _BB * _O
_RE = _BB * _EPS
_NH = 4
_BH = _BB // _NH
_RH = _BH * _O


def _sel_matrices():
    s0 = np.zeros((_EPS, _O), np.float32)
    c0 = np.zeros((_EPS, _O), np.float32)
    e = 0
    for i in range(_O):
        for j in range(_O):
            if i == j:
                continue
            s0[e, i] = 1.0
            c0[e, j] = 1.0
            e += 1
    eye = np.eye(_BH, dtype=np.float32)
    S = np.concatenate([np.kron(eye, s0), np.kron(eye, c0)], axis=1)
    Ragg = np.kron(eye, s0.T)
    return S, Ragg


_S_NP, _RAGG_NP = _sel_matrices()


def _c4_flat(W):
    g = jnp.arange(4)[:, None]
    h = jnp.arange(4)[None, :]
    Wfull = W[(g - h) % 4]
    Wt = jnp.transpose(Wfull, (1, 2, 0, 3))
    return Wt.reshape(4 * W.shape[1], 4 * W.shape[2])


def _ln_relu(y, gamma, beta):
    outs = []
    for gi in range(4):
        c = y[:, gi * _HID:(gi + 1) * _HID]
        mu = jnp.mean(c, axis=1, keepdims=True)
        d = c - mu
        var = jnp.mean(d * d, axis=1, keepdims=True)
        outs.append(d * jax.lax.rsqrt(var + 1e-5) * gamma + beta)
    return jnp.maximum(jnp.concatenate(outs, axis=1), 0.0)


def _dot(a, b):
    return jnp.dot(a.astype(jnp.bfloat16), b,
                   preferred_element_type=jnp.float32)


def _chain(x, M, sel, ragg, w1s, w1t, b1, w2, b2, ge, bne, w3, b3,
           wn1o, wav, wn1a, bn1, wn2, bn2, gn, bnn, wn3, bn3):
    A = _dot(x, w1s)
    T = _dot(x, w1t)
    AT = jnp.concatenate([A, T], axis=0)

    e = _dot(sel, AT) + b1
    e = jnp.maximum(e, 0.0)

    e = _dot(e, w2) + b2
    e = _ln_relu(e, ge, bne)
    e = _dot(e, w3) + b3

    agg = _dot(ragg, e)

    n = _dot(x, wn1o) + _dot(M, wav) + _dot(agg, wn1a) + bn1
    n = jnp.maximum(n, 0.0)
    n = _dot(n, wn2) + bn2
    n = _ln_relu(n, gn, bnn)
    return _dot(n, wn3) + bn3


def _body(x_ref, act_ref, *refs):
    out_ref = refs[-1]
    ws = tuple(r[...] for r in refs[:-1])

    act = act_ref[0, 0, :].reshape(_R, 1)
    rr = jax.lax.broadcasted_iota(jnp.int32, (_R, 4), 0)
    hh = jax.lax.broadcasted_iota(jnp.int32, (_R, 4), 1)
    M = (act == 4 * (rr % _O) + hh).astype(jnp.float32)

    for h in range(_NH):
        r0 = h * _RH
        out_ref[r0:r0 + _RH, :] = _chain(
            x_ref[r0:r0 + _RH, :], M[r0:r0 + _RH, :], *ws)


def kernel(states, action, We1, be1, We2, be2, ge, bne, We3, be3,
           Wn1, bn1, Wn2, bn2, gn, bnn, Wn3, bn3):
    x = states.reshape(_B * _O, 4 * _OBS).astype(jnp.bfloat16)
    nblk = _B // _BB
    act = jnp.repeat(action.astype(jnp.int32), _O).reshape(nblk, 1, _R)

    bf16 = jnp.bfloat16
    S = jnp.asarray(_S_NP, dtype=bf16)
    Ragg = jnp.asarray(_RAGG_NP, dtype=bf16)
    W1s = _c4_flat(We1[:, :_OBS, :]).astype(bf16)
    W1t = _c4_flat(We1[:, _OBS:, :]).astype(bf16)
    W2 = _c4_flat(We2).astype(bf16)
    W3 = _c4_flat(We3).astype(bf16)
    Wn1o = _c4_flat(Wn1[:, :_OBS, :]).astype(bf16)
    Wav = _c4_flat(Wn1[:, _OBS:_OBS + 1, :]).astype(bf16)
    Wn1a = _c4_flat(Wn1[:, _OBS + 1:, :]).astype(bf16)
    Wn2f = _c4_flat(Wn2).astype(bf16)
    Wn3f = _c4_flat(Wn3).astype(bf16)

    b1 = jnp.tile(be1, 4).reshape(1, _F)
    b2 = jnp.tile(be2, 4).reshape(1, _F)
    b3 = jnp.tile(be3, 4).reshape(1, _F)
    bn1r = jnp.tile(bn1, 4).reshape(1, _F)
    bn2r = jnp.tile(bn2, 4).reshape(1, _F)
    bn3r = jnp.tile(bn3, 4).reshape(1, 4 * _OBS)
    ge2 = ge.reshape(1, _HID)
    bne2 = bne.reshape(1, _HID)
    gn2 = gn.reshape(1, _HID)
    bnn2 = bnn.reshape(1, _HID)

    def const_spec(a):
        nd = a.ndim
        return pl.BlockSpec(a.shape, lambda i, _nd=nd: (0,) * _nd)

    weights = (S, Ragg, W1s, W1t, b1, W2, b2, ge2, bne2, W3, b3,
               Wn1o, Wav, Wn1a, bn1r, Wn2f, bn2r, gn2, bnn2, Wn3f, bn3r)

    out = pl.pallas_call(
        _body,
        grid=(nblk,),
        in_specs=[
            pl.BlockSpec((_R, 4 * _OBS), lambda i: (i, 0)),
            pl.BlockSpec((1, 1, _R), lambda i: (i, 0, 0)),
        ] + [const_spec(w) for w in weights],
        out_specs=pl.BlockSpec((_R, 4 * _OBS), lambda i: (i, 0)),
        out_shape=jax.ShapeDtypeStruct((_B * _O, 4 * _OBS), jnp.float32),
        compiler_params=pltpu.CompilerParams(
            dimension_semantics=("arbitrary",)),
    )(x, act, *weights)

    return out.reshape(_B, _O, 4, _OBS)
